# baseline (speedup 1.0000x reference)
import jax
import jax.numpy as jnp
from jax.experimental import pallas as pl
from jax.experimental.pallas import tpu as pltpu
from jax.experimental.pallas import tpu_sc as plsc

B, L, C = 64, 128, 256
E0 = 4096
H, HD = 4, 32
HID = H * HD
NEG = 0.2
EPS = 1e-5
NEG_BIG = -1e30
RT = 64


SC_NC = 2
SC_NS = 16
EPW = E0 // (SC_NC * SC_NS)


def _sc_count_kernel(src_hbm, dst_hbm, zeros_hbm, out_hbm, src_v, dst_v,
                     slab):
    c = jax.lax.axis_index("c")
    s = jax.lax.axis_index("s")
    wid = c * SC_NS + s
    lane = jax.lax.iota(jnp.int32, 16)
    ones16 = jnp.ones((16,), jnp.float32)
    pltpu.sync_copy(zeros_hbm, slab)
    base = wid * EPW
    pltpu.sync_copy(src_hbm.at[pl.ds(base, EPW)], src_v)
    pltpu.sync_copy(dst_hbm.at[pl.ds(base, EPW)], dst_v)
    for k in range(EPW // 16):
        sv = src_v[pl.ds(k * 16, 16)]
        dv = dst_v[pl.ds(k * 16, 16)]
        valid = sv != dv
        for j in range(16):
            plsc.addupdate_scatter(slab, [dv, sv], ones16,
                                   mask=valid & (lane == j))
    pltpu.sync_copy(slab, out_hbm.at[wid])


def _reduce_kernel(planes_ref, m_ref):
    m = jnp.sum(planes_ref[...], axis=0)
    eye = (jax.lax.broadcasted_iota(jnp.int32, (C, C), 0)
           == jax.lax.broadcasted_iota(jnp.int32, (C, C), 1))
    m_ref[...] = m + eye.astype(jnp.float32)


def _gat_layer(h, m1, mask, Wl, bl, Wr, br, att, att_s, bg):
    xl = jnp.dot(h, Wl, preferred_element_type=jnp.float32) + bl
    xr = jnp.dot(h, Wr, preferred_element_type=jnp.float32) + br
    xlT = xl.T
    outs = []
    attn_sum = jnp.zeros((C, C), jnp.float32)
    c_lin = 0.5 * (1.0 + NEG)
    c_abs = 0.5 * (1.0 - NEG)
    for hh in range(H):
        xrh = xr[:, hh * HD:(hh + 1) * HD]
        zl = xlT[hh * HD:(hh + 1) * HD]
        atth = att[hh].reshape(1, HD)
        rowt = c_lin * jnp.sum(xrh * atth, axis=1, keepdims=True)
        colt = c_lin * jnp.sum(zl * atth.T, axis=0, keepdims=True)
        logit = rowt + colt
        for d in range(HD):
            w = att_s[hh, d] * c_abs
            logit = logit + w * jnp.abs(xrh[:, d:d + 1] + zl[d:d + 1, :])
        logit = jnp.where(mask, logit, NEG_BIG)
        amax = jnp.max(logit, axis=1, keepdims=True)
        ex = jnp.where(mask, jnp.exp(logit - amax), 0.0)
        wex = m1 * ex
        den = jnp.sum(wex, axis=1, keepdims=True)
        A = wex / (den + 1e-16)
        attn_sum = attn_sum + A
        outs.append(jnp.dot(A, xl[:, hh * HD:(hh + 1) * HD],
                            preferred_element_type=jnp.float32))
    nh = jnp.concatenate(outs, axis=1) + bg
    return nh, attn_sum * (1.0 / H)


def _ln(h, g, b):
    mu = jnp.mean(h, axis=-1, keepdims=True)
    v = jnp.mean((h - mu) ** 2, axis=-1, keepdims=True)
    return (h - mu) * jax.lax.rsqrt(v + EPS) * g + b


def _main_kernel(x_ref, m1_ref,
                 Wemb_ref, bemb_ref,
                 Wl0_ref, bl0_ref, Wr0_ref, br0_ref, att0_ref, bg0_ref,
                 g0_ref, be0_ref,
                 Wl1_ref, bl1_ref, Wr1_ref, br1_ref, att1_ref, bg1_ref,
                 g1_ref, be1_ref,
                 Wproj_ref, bproj_ref, att0s_ref, att1s_ref,
                 out_ref, attn_ref):
    xb = x_ref[0]
    m1 = m1_ref[...]
    mask = m1 > 0
    h = jax.lax.dot_general(xb, Wemb_ref[...], (((0,), (0,)), ((), ())),
                            preferred_element_type=jnp.float32) + bemb_ref[...]

    layer_params = (
        (Wl0_ref, bl0_ref, Wr0_ref, br0_ref, att0_ref, att0s_ref, bg0_ref,
         g0_ref, be0_ref),
        (Wl1_ref, bl1_ref, Wr1_ref, br1_ref, att1_ref, att1s_ref, bg1_ref,
         g1_ref, be1_ref),
    )
    attn = None
    for (Wl, bl, Wr, br, att, att_s, bg, g, be) in layer_params:
        nh, attn = _gat_layer(h, m1, mask, Wl[...], bl[...], Wr[...], br[...],
                              att[...], att_s, bg[...])
        nh = jnp.where(nh > 0, nh, jnp.exp(jnp.minimum(nh, 0.0)) - 1.0)
        h = _ln(h + nh, g[...], be[...])

    out = jnp.dot(h, Wproj_ref[...], preferred_element_type=jnp.float32)
    out = out + bproj_ref[...].T
    out_ref[0] = out.T
    attn_ref[0] = attn


def kernel(x, edge_index, W_emb, b_emb, Wl0, bl0, Wr0, br0, att0, bg0, g0,
           be0, Wl1, bl1, Wr1, br1, att1, bg1, g1, be1, W_proj, b_proj):
    planes = pl.kernel(
        _sc_count_kernel,
        out_type=jax.ShapeDtypeStruct((SC_NC * SC_NS, C, C), jnp.float32),
        mesh=plsc.VectorSubcoreMesh(core_axis_name="c", subcore_axis_name="s"),
        compiler_params=pltpu.CompilerParams(needs_layout_passes=False),
        scratch_types=[
            pltpu.VMEM((EPW,), jnp.int32),
            pltpu.VMEM((EPW,), jnp.int32),
            pltpu.VMEM((C, C), jnp.float32),
        ],
    )(edge_index[0], edge_index[1], jnp.zeros((C, C), jnp.float32))
    m1 = pl.pallas_call(
        _reduce_kernel,
        out_shape=jax.ShapeDtypeStruct((C, C), jnp.float32),
    )(planes)

    r2 = lambda v: v.reshape(1, -1)
    const = lambda shape: pl.BlockSpec(shape, lambda b: (0,) * len(shape))
    in_specs = [
        pl.BlockSpec((1, L, C), lambda b: (b, 0, 0)),
        const((C, C)),
        const((L, HID)), const((1, HID)),
    ]
    for _ in range(2):
        in_specs += [
            const((HID, HID)), const((1, HID)),
            const((HID, HID)), const((1, HID)),
            const((H, HD)), const((1, HID)),
            const((1, HID)), const((1, HID)),
        ]
    in_specs += [const((HID, L)), const((L, 1))]
    in_specs += [pl.BlockSpec(memory_space=pltpu.SMEM)] * 2

    out, attn = pl.pallas_call(
        _main_kernel,
        grid=(B,),
        compiler_params=pltpu.CompilerParams(
            dimension_semantics=("parallel",)),
        in_specs=in_specs,
        out_specs=[
            pl.BlockSpec((1, L, C), lambda b: (b, 0, 0)),
            pl.BlockSpec((1, C, C), lambda b: (b, 0, 0)),
        ],
        out_shape=[
            jax.ShapeDtypeStruct((B, L, C), jnp.float32),
            jax.ShapeDtypeStruct((B, C, C), jnp.float32),
        ],
    )(x, m1, W_emb, r2(b_emb),
      Wl0, r2(bl0), Wr0, r2(br0), att0, r2(bg0), r2(g0), r2(be0),
      Wl1, r2(bl1), Wr1, r2(br1), att1, r2(bg1), r2(g1), r2(be1),
      W_proj, b_proj.reshape(L, 1), att0, att1)
    return out, attn


# speedup vs baseline: 1.0273x; 1.0273x over previous
import jax
import jax.numpy as jnp
from jax.experimental import pallas as pl
from jax.experimental.pallas import tpu as pltpu
from jax.experimental.pallas import tpu_sc as plsc

B, L, C = 64, 128, 256
E0 = 4096
H, HD = 4, 32
HID = H * HD
NEG = 0.2
EPS = 1e-5
NEG_BIG = -1e30
BPP = 2


SC_NC = 2
SC_NS = 16
EPW = E0 // (SC_NC * SC_NS)


def _sc_count_kernel(src_hbm, dst_hbm, zeros_hbm, out_hbm, src_v, dst_v,
                     slab):
    c = jax.lax.axis_index("c")
    s = jax.lax.axis_index("s")
    wid = c * SC_NS + s
    lane = jax.lax.iota(jnp.int32, 16)
    ones16 = jnp.ones((16,), jnp.float32)
    pltpu.sync_copy(zeros_hbm, slab)
    base = wid * EPW
    pltpu.sync_copy(src_hbm.at[pl.ds(base, EPW)], src_v)
    pltpu.sync_copy(dst_hbm.at[pl.ds(base, EPW)], dst_v)
    for k in range(EPW // 16):
        sv = src_v[pl.ds(k * 16, 16)]
        dv = dst_v[pl.ds(k * 16, 16)]
        valid = sv != dv
        for j in range(16):
            plsc.addupdate_scatter(slab, [dv, sv], ones16,
                                   mask=valid & (lane == j))
    pltpu.sync_copy(slab, out_hbm.at[wid])


def _reduce_kernel(planes_ref, m_ref):
    m = jnp.sum(planes_ref[...], axis=0)
    eye = (jax.lax.broadcasted_iota(jnp.int32, (C, C), 0)
           == jax.lax.broadcasted_iota(jnp.int32, (C, C), 1))
    m_ref[...] = m + eye.astype(jnp.float32)


def _gat_layer(h, m1, mask, Wl, bl, Wr, br, att, att_s, bg):
    xl = jnp.dot(h, Wl, preferred_element_type=jnp.float32) + bl
    xr = jnp.dot(h, Wr, preferred_element_type=jnp.float32) + br
    xlT = xl.T
    outs = []
    attn_sum = jnp.zeros((C, C), jnp.float32)
    c_lin = 0.5 * (1.0 + NEG)
    c_abs = 0.5 * (1.0 - NEG)
    for hh in range(H):
        xrh = xr[:, hh * HD:(hh + 1) * HD]
        zl = xlT[hh * HD:(hh + 1) * HD]
        atth = att[hh].reshape(1, HD)
        rowt = c_lin * jnp.sum(xrh * atth, axis=1, keepdims=True)
        colt = c_lin * jnp.sum(zl * atth.T, axis=0, keepdims=True)
        logit = rowt + colt
        for d in range(HD):
            w = att_s[hh, d] * c_abs
            logit = logit + w * jnp.abs(xrh[:, d:d + 1] + zl[d:d + 1, :])
        logit = jnp.where(mask, logit, NEG_BIG)
        amax = jnp.max(logit, axis=1, keepdims=True)
        ex = jnp.where(mask, jnp.exp(logit - amax), 0.0)
        wex = m1 * ex
        den = jnp.sum(wex, axis=1, keepdims=True)
        A = wex / (den + 1e-16)
        attn_sum = attn_sum + A
        outs.append(jnp.dot(A, xl[:, hh * HD:(hh + 1) * HD],
                            preferred_element_type=jnp.float32))
    nh = jnp.concatenate(outs, axis=1) + bg
    return nh, attn_sum * (1.0 / H)


def _ln(h, g, b):
    mu = jnp.mean(h, axis=-1, keepdims=True)
    v = jnp.mean((h - mu) ** 2, axis=-1, keepdims=True)
    return (h - mu) * jax.lax.rsqrt(v + EPS) * g + b


def _main_kernel(x_ref, m1_ref,
                 Wemb_ref, bemb_ref,
                 Wl0_ref, bl0_ref, Wr0_ref, br0_ref, att0_ref, bg0_ref,
                 g0_ref, be0_ref,
                 Wl1_ref, bl1_ref, Wr1_ref, br1_ref, att1_ref, bg1_ref,
                 g1_ref, be1_ref,
                 Wproj_ref, bproj_ref, att0s_ref, att1s_ref,
                 out_ref, attn_ref):
    m1 = m1_ref[...]
    mask = m1 > 0
    layer_params = (
        (Wl0_ref, bl0_ref, Wr0_ref, br0_ref, att0_ref, att0s_ref, bg0_ref,
         g0_ref, be0_ref),
        (Wl1_ref, bl1_ref, Wr1_ref, br1_ref, att1_ref, att1s_ref, bg1_ref,
         g1_ref, be1_ref),
    )
    for b in range(BPP):
        xb = x_ref[b]
        h = jax.lax.dot_general(xb, Wemb_ref[...], (((0,), (0,)), ((), ())),
                                preferred_element_type=jnp.float32)
        h = h + bemb_ref[...]
        attn = None
        for (Wl, bl, Wr, br, att, att_s, bg, g, be) in layer_params:
            nh, attn = _gat_layer(h, m1, mask, Wl[...], bl[...], Wr[...],
                                  br[...], att[...], att_s, bg[...])
            nh = jnp.where(nh > 0, nh, jnp.exp(jnp.minimum(nh, 0.0)) - 1.0)
            h = _ln(h + nh, g[...], be[...])

        out = jnp.dot(h, Wproj_ref[...], preferred_element_type=jnp.float32)
        out = out + bproj_ref[...].T
        out_ref[b] = out.T
        attn_ref[b] = attn


def kernel(x, edge_index, W_emb, b_emb, Wl0, bl0, Wr0, br0, att0, bg0, g0,
           be0, Wl1, bl1, Wr1, br1, att1, bg1, g1, be1, W_proj, b_proj):
    planes = pl.kernel(
        _sc_count_kernel,
        out_type=jax.ShapeDtypeStruct((SC_NC * SC_NS, C, C), jnp.float32),
        mesh=plsc.VectorSubcoreMesh(core_axis_name="c", subcore_axis_name="s"),
        compiler_params=pltpu.CompilerParams(needs_layout_passes=False),
        scratch_types=[
            pltpu.VMEM((EPW,), jnp.int32),
            pltpu.VMEM((EPW,), jnp.int32),
            pltpu.VMEM((C, C), jnp.float32),
        ],
    )(edge_index[0], edge_index[1], jnp.zeros((C, C), jnp.float32))
    m1 = pl.pallas_call(
        _reduce_kernel,
        out_shape=jax.ShapeDtypeStruct((C, C), jnp.float32),
    )(planes)

    r2 = lambda v: v.reshape(1, -1)
    const = lambda shape: pl.BlockSpec(shape, lambda b: (0,) * len(shape))
    in_specs = [
        pl.BlockSpec((BPP, L, C), lambda b: (b, 0, 0)),
        const((C, C)),
        const((L, HID)), const((1, HID)),
    ]
    for _ in range(2):
        in_specs += [
            const((HID, HID)), const((1, HID)),
            const((HID, HID)), const((1, HID)),
            const((H, HD)), const((1, HID)),
            const((1, HID)), const((1, HID)),
        ]
    in_specs += [const((HID, L)), const((L, 1))]
    in_specs += [pl.BlockSpec(memory_space=pltpu.SMEM)] * 2

    out, attn = pl.pallas_call(
        _main_kernel,
        grid=(B // BPP,),
        compiler_params=pltpu.CompilerParams(
            dimension_semantics=("parallel",)),
        in_specs=in_specs,
        out_specs=[
            pl.BlockSpec((BPP, L, C), lambda b: (b, 0, 0)),
            pl.BlockSpec((BPP, C, C), lambda b: (b, 0, 0)),
        ],
        out_shape=[
            jax.ShapeDtypeStruct((B, L, C), jnp.float32),
            jax.ShapeDtypeStruct((B, C, C), jnp.float32),
        ],
    )(x, m1, W_emb, r2(b_emb),
      Wl0, r2(bl0), Wr0, r2(br0), att0, r2(bg0), r2(g0), r2(be0),
      Wl1, r2(bl1), Wr1, r2(br1), att1, r2(bg1), r2(g1), r2(be1),
      W_proj, b_proj.reshape(L, 1), att0, att1)
    return out, attn
